# baseline (speedup 1.0000x reference)
import functools
import numpy as np
import jax
import jax.numpy as jnp
from jax import lax
from jax.experimental import pallas as pl
from jax.experimental.pallas import tpu as pltpu
from jax.experimental.pallas import tpu_sc as plsc

_N = 100
_LVLS = 3
_D0 = 16
_C1 = 16
_C2 = 32
_EDGE_P = 0.06


def _structure():
    rng = np.random.RandomState(0)
    A = rng.rand(_N, _N) < _EDGE_P
    A = np.triu(A, 1)
    A = A | A.T
    nbhd1 = [sorted(set([v]) | set(np.nonzero(A[v])[0].tolist()))
             for v in range(_N)]
    rf = [[[v] for v in range(_N)]]
    for _ in range(1, _LVLS):
        prev = rf[-1]
        cur = []
        for v in range(_N):
            s = set()
            for w in nbhd1[v]:
                s.update(prev[w])
            cur.append(sorted(s))
        rf.append(cur)
    return nbhd1, rf


_NBHD1, _RF = _structure()
_OUT_V = list(range(_LVLS))
_W_NEED = sorted(set().union(*[set(_NBHD1[v]) for v in _OUT_V]))
_K1 = {w: len(_RF[1][w]) for w in _W_NEED}
_K2 = {v: len(_RF[2][v]) for v in _OUT_V}

_T1 = sum(k * k for k in _K1.values())
_T2 = sum(K * K for K in _K2.values())
_NW = 16
_R1W = 96
_R2W = 192
_T1P = 16 * _R1W
_T2P = _NW * _R2W
_E1 = _T1P * 16
_E1W = _R1W * 16
_E2W = _R2W * 16
_FAN = 7
_SENT = _E1

_TOFF1 = {}
_o = 0
for _w in _W_NEED:
    _TOFF1[_w] = _o
    _o += _K1[_w] * _K1[_w]
_TOFF2 = {}
_o = 0
for _v in _OUT_V:
    _TOFF2[_v] = _o
    _o += _K2[_v] * _K2[_v]

_XI = np.zeros((_E1,), np.int32)
_XMF = np.zeros((_E1,), np.float32)
_AI = np.zeros((_E1,), np.int32)
for _w in _W_NEED:
    _k = _K1[_w]
    _S = _RF[1][_w]
    _base = _TOFF1[_w] * 16
    for _m in range(16 * _k * _k):
        _e = _base + _m
        _c, _rem = divmod(_m, _k * _k)
        _i, _j = divmod(_rem, _k)
        _AI[_e] = _S[_i] * _N + _S[_j]
        if _i == _j:
            _XI[_e] = _S[_i] * 16 + _c
            _XMF[_e] = 1.0

_A2I = np.zeros((_T2P * 16,), np.int32)
_SRC = np.full((_FAN, _T2P * 16), _SENT, np.int32)
_CNT = np.zeros((_T2P * 16,), np.int32)
for _v in _OUT_V:
    _K = _K2[_v]
    _S2 = _RF[2][_v]
    _pos2 = {u: i for i, u in enumerate(_S2)}
    _b2 = _TOFF2[_v] * 16
    for _m in range(16 * _K * _K):
        _e = _b2 + _m
        _c, _rem = divmod(_m, _K * _K)
        _I, _J = divmod(_rem, _K)
        _A2I[_e] = _S2[_I] * _N + _S2[_J]
    for _w in _NBHD1[_v]:
        _k = _K1[_w]
        _S1 = _RF[1][_w]
        for _c in range(16):
            for _il in range(_k):
                for _jl in range(_k):
                    _m = _c * _K * _K + _pos2[_S1[_il]] * _K + _pos2[_S1[_jl]]
                    _e = _b2 + _m
                    _SRC[_CNT[_e], _e] = (_TOFF1[_w] * 16
                                          + _c * _k * _k + _il * _k + _jl)
                    _CNT[_e] += 1

_CH2 = np.full((_T2P, 32), 32, np.int32)
for _v in _OUT_V:
    _K = _K2[_v]
    for _rl in range(_K * _K):
        _row = _TOFF2[_v] + _rl
        for _oo in range(32):
            _CH2[_row, _oo] = (_rl * 32 + _oo) // (_K * _K)

_XI_O = 0
_XM_O = _E1W
_AI_O = 2 * _E1W
_A2_O = 3 * _E1W
_SR_O = _A2_O + _E2W
_CH_O = _SR_O + _FAN * _E2W
_RTBL = _CH_O + _R2W * 32

_TBL = np.zeros((_NW, _RTBL), np.int32)
for _sid in range(_NW):
    _sl1 = slice(_sid * _E1W, (_sid + 1) * _E1W)
    _sl2 = slice(_sid * _E2W, (_sid + 1) * _E2W)
    _TBL[_sid, _XI_O:_XI_O + _E1W] = _XI[_sl1]
    _TBL[_sid, _XM_O:_XM_O + _E1W] = _XMF[_sl1].view(np.int32)
    _TBL[_sid, _AI_O:_AI_O + _E1W] = _AI[_sl1]
    _TBL[_sid, _A2_O:_A2_O + _E2W] = _A2I[_sl2]
    for _f in range(_FAN):
        _TBL[_sid, _SR_O + _f * _E2W:_SR_O + (_f + 1) * _E2W] = _SRC[_f, _sl2]
    _cht = np.empty((_R2W // 16, 32, 16), np.int32)
    for _b in range(_R2W // 16):
        for _oo in range(32):
            for _i in range(16):
                _cht[_b, _oo, _i] = _CH2[_sid * _R2W + _b * 16 + _i, _oo]
    _TBL[_sid, _CH_O:_CH_O + _R2W * 32] = _cht.ravel()

_f32 = jnp.float32


def _sc_body(x_hbm, adj_hbm, scal_hbm, b1_hbm, w1_hbm, b2_hbm, w2_hbm,
             fcw_hbm, tbl_hbm, out_hbm, g_hbm,
             xv, adjv, scalv, b1r, w1r, b2r, w2r, fcwr, tblv,
             h1c, h1ext, fl2, sacc, sred, outv, gv, dsem, sh_h1, sh_s):
    cid = lax.axis_index("c")
    sid = lax.axis_index("s")
    iot = lax.iota(jnp.int32, 16)

    def core0_work():
        copies = [
            pltpu.make_async_copy(x_hbm, xv, dsem),
            pltpu.make_async_copy(adj_hbm, adjv, dsem),
            pltpu.make_async_copy(scal_hbm, scalv, dsem),
            pltpu.make_async_copy(b1_hbm, b1r, dsem),
            pltpu.make_async_copy(w1_hbm, w1r, dsem),
            pltpu.make_async_copy(b2_hbm, b2r, dsem),
            pltpu.make_async_copy(w2_hbm, w2r, dsem),
            pltpu.make_async_copy(fcw_hbm, fcwr, dsem),
            pltpu.make_async_copy(tbl_hbm.at[sid], tblv, dsem),
        ]
        for cp in copies:
            cp.start()
        for cp in copies:
            cp.wait()

        misc = scalv[...]
        lam1 = misc[0]
        lam2 = misc[1]
        fcb = misc[2]
        b1v = b1r[...]
        w1v = [w1r[pl.ds(o * 16, 16)] for o in range(16)]

        def phase_a(b, carry):
            base = b * 256
            cols = []
            for c in range(16):
                ei = base + c + iot * 16
                xi = plsc.load_gather(tblv, [_XI_O + ei])
                xm = plsc.bitcast(plsc.load_gather(tblv, [_XM_O + ei]), _f32)
                ai = plsc.load_gather(tblv, [_AI_O + ei])
                xval = plsc.load_gather(xv, [xi])
                aval = plsc.load_gather(adjv, [ai])
                cols.append(xm * xval + lam1 * aval)
            for o in range(16):
                acc = cols[0] * w1v[o][0]
                for c in range(1, 16):
                    acc = acc + cols[c] * w1v[o][c]
                val = jnp.maximum(acc + b1v[o], 0.0)
                plsc.store_scatter(h1c, [base + o + iot * 16], val)
            return carry

        lax.fori_loop(0, _R1W // 16, phase_a, None)

        pltpu.sync_copy(h1c, sh_h1.at[pl.ds(sid * _E1W, _E1W)])
        plsc.subcore_barrier()
        pltpu.sync_copy(sh_h1, h1ext.at[pl.ds(0, _E1)])
        h1ext[pl.ds(_E1, 16)] = jnp.zeros((16,), _f32)

        def phase_b(g, carry):
            e = g * 16 + iot
            a2 = plsc.load_gather(tblv, [_A2_O + e])
            acc = lam2 * plsc.load_gather(adjv, [a2])
            for f in range(_FAN):
                si = plsc.load_gather(tblv, [_SR_O + f * _E2W + e])
                acc = acc + plsc.load_gather(h1ext, [si])
            plsc.store_scatter(fl2, [e], acc)
            return carry

        lax.fori_loop(0, _R2W, phase_b, None)

        sacc[pl.ds(0, 16)] = jnp.zeros((16,), _f32)
        sacc[pl.ds(16, 16)] = jnp.zeros((16,), _f32)
        sacc[pl.ds(32, 16)] = jnp.zeros((16,), _f32)

        b2v = [b2r[pl.ds(0, 16)], b2r[pl.ds(16, 16)]]
        w2v = [w2r[pl.ds(o * 16, 16)] for o in range(32)]

        def phase_c(b, carry):
            base = b * 256
            cols = []
            for c in range(16):
                cols.append(plsc.load_gather(fl2, [base + c + iot * 16]))
            for o in range(32):
                acc = cols[0] * w2v[o][0]
                for c in range(1, 16):
                    acc = acc + cols[c] * w2v[o][c]
                val = jnp.maximum(acc + b2v[o // 16][o % 16], 0.0)
                sidx = plsc.load_gather(tblv,
                                        [_CH_O + b * 512 + o * 16 + iot])
                plsc.addupdate_scatter(sacc, [sidx], val)
            return carry

        lax.fori_loop(0, _R2W // 16, phase_c, None)

        pltpu.sync_copy(sacc, sh_s.at[pl.ds(sid * 48, 48)])
        plsc.subcore_barrier()

        def final_reduce():
            pltpu.sync_copy(sh_s, sred)
            s0 = sred[pl.ds(0, 16)]
            s1 = sred[pl.ds(16, 16)]
            for i in range(1, 16):
                s0 = s0 + sred[pl.ds(i * 48, 16)]
                s1 = s1 + sred[pl.ds(i * 48 + 16, 16)]
            prod = s0 * fcwr[pl.ds(0, 16)] + s1 * fcwr[pl.ds(16, 16)]
            tot = jnp.sum(prod) + fcb
            outv[...] = jnp.zeros((16,), _f32) + tot
            gv[pl.ds(0, 16)] = s0
            gv[pl.ds(16, 16)] = s1
            pltpu.sync_copy(outv, out_hbm)
            pltpu.sync_copy(gv, g_hbm)

        pl.when(sid == 0)(final_reduce)

    pl.when(cid == 0)(core0_work)


def kernel(x, adj, W1, b1, W2, b2, adj_lambda_1, adj_lambda_2, fc_w, fc_b):
    scal = jnp.concatenate([
        adj_lambda_1.reshape(-1), adj_lambda_2.reshape(-1),
        fc_b.reshape(-1), jnp.zeros((13,), _f32),
    ])
    mesh = plsc.VectorSubcoreMesh(core_axis_name="c", subcore_axis_name="s")
    sc = functools.partial(
        pl.kernel, _sc_body, mesh=mesh,
        compiler_params=pltpu.CompilerParams(needs_layout_passes=False),
        out_type=[jax.ShapeDtypeStruct((16,), _f32),
                  jax.ShapeDtypeStruct((32,), _f32)],
        scratch_types=[
            pltpu.VMEM((1600,), _f32),
            pltpu.VMEM((_N * _N,), _f32),
            pltpu.VMEM((16,), _f32),
            pltpu.VMEM((16,), _f32),
            pltpu.VMEM((256,), _f32),
            pltpu.VMEM((32,), _f32),
            pltpu.VMEM((512,), _f32),
            pltpu.VMEM((32,), _f32),
            pltpu.VMEM((_RTBL,), jnp.int32),
            pltpu.VMEM((_E1W,), _f32),
            pltpu.VMEM((_E1 + 16,), _f32),
            pltpu.VMEM((_E2W,), _f32),
            pltpu.VMEM((48,), _f32),
            pltpu.VMEM((16 * 48,), _f32),
            pltpu.VMEM((16,), _f32),
            pltpu.VMEM((32,), _f32),
            pltpu.SemaphoreType.DMA,
            pltpu.VMEM_SHARED((_E1,), _f32),
            pltpu.VMEM_SHARED((16 * 48,), _f32),
        ],
    )()
    out16, g32 = sc(x.reshape(-1), adj.reshape(-1), scal,
                    b1, W1.reshape(-1), b2, W2.reshape(-1),
                    fc_w.reshape(-1), jnp.asarray(_TBL))
    return out16[0:1].reshape(1, 1), g32.reshape(1, _C2)


# speedup vs baseline: 1.1689x; 1.1689x over previous
import functools
import numpy as np
import jax
import jax.numpy as jnp
from jax import lax
from jax.experimental import pallas as pl
from jax.experimental.pallas import tpu as pltpu
from jax.experimental.pallas import tpu_sc as plsc

_N = 100
_LVLS = 3
_D0 = 16
_C1 = 16
_C2 = 32
_EDGE_P = 0.06


def _structure():
    rng = np.random.RandomState(0)
    A = rng.rand(_N, _N) < _EDGE_P
    A = np.triu(A, 1)
    A = A | A.T
    nbhd1 = [sorted(set([v]) | set(np.nonzero(A[v])[0].tolist()))
             for v in range(_N)]
    rf = [[[v] for v in range(_N)]]
    for _ in range(1, _LVLS):
        prev = rf[-1]
        cur = []
        for v in range(_N):
            s = set()
            for w in nbhd1[v]:
                s.update(prev[w])
            cur.append(sorted(s))
        rf.append(cur)
    return nbhd1, rf


_NBHD1, _RF = _structure()
_OUT_V = list(range(_LVLS))
_W_NEED = sorted(set().union(*[set(_NBHD1[v]) for v in _OUT_V]))
_K1 = {w: len(_RF[1][w]) for w in _W_NEED}
_K2 = {v: len(_RF[2][v]) for v in _OUT_V}

_T1 = sum(k * k for k in _K1.values())
_T2 = sum(K * K for K in _K2.values())
_NW = 32
_R1W = 96
_R2W = 96
_T1P = 16 * _R1W
_T2P = _NW * _R2W
_E1 = _T1P * 16
_E1W = _R1W * 16
_E2W = _R2W * 16
_SENT = _E1
_MAXS = 800
_DUMP2 = _E2W

_TOFF1 = {}
_o = 0
for _w in _W_NEED:
    _TOFF1[_w] = _o
    _o += _K1[_w] * _K1[_w]
_TOFF2 = {}
_o = 0
for _v in _OUT_V:
    _TOFF2[_v] = _o
    _o += _K2[_v] * _K2[_v]

_XI = np.zeros((_E1,), np.int32)
_XMF = np.zeros((_E1,), np.float32)
_AI = np.zeros((_E1,), np.int32)
for _w in _W_NEED:
    _k = _K1[_w]
    _S = _RF[1][_w]
    _base = _TOFF1[_w] * 16
    for _m in range(16 * _k * _k):
        _e = _base + _m
        _c, _rem = divmod(_m, _k * _k)
        _i, _j = divmod(_rem, _k)
        _AI[_e] = _S[_i] * _N + _S[_j]
        if _i == _j:
            _XI[_e] = _S[_i] * 16 + _c
            _XMF[_e] = 1.0

_A2I = np.zeros((_T2P * 16,), np.int32)
_PAIRS = [[] for _ in range(_NW)]
for _v in _OUT_V:
    _K = _K2[_v]
    _S2 = _RF[2][_v]
    _pos2 = {u: i for i, u in enumerate(_S2)}
    _b2 = _TOFF2[_v] * 16
    for _m in range(16 * _K * _K):
        _e = _b2 + _m
        _c, _rem = divmod(_m, _K * _K)
        _I, _J = divmod(_rem, _K)
        _A2I[_e] = _S2[_I] * _N + _S2[_J]
    for _w in _NBHD1[_v]:
        _k = _K1[_w]
        _S1 = _RF[1][_w]
        for _c in range(16):
            for _il in range(_k):
                for _jl in range(_k):
                    _m = _c * _K * _K + _pos2[_S1[_il]] * _K + _pos2[_S1[_jl]]
                    _e = _b2 + _m
                    _src = _TOFF1[_w] * 16 + _c * _k * _k + _il * _k + _jl
                    _PAIRS[_e // _E2W].append((_src, _e % _E2W))
                    _CNTCHK = None

_SLS = np.full((_NW, _MAXS), _SENT, np.int32)
_SLD = np.full((_NW, _MAXS), _DUMP2, np.int32)
for _t in range(_NW):
    groups = []
    for _src, _d in _PAIRS[_t]:
        for _grp in groups:
            if _d not in _grp[0] and len(_grp[1]) < 16:
                _grp[0].add(_d)
                _grp[1].append((_src, _d))
                break
        else:
            groups.append(({_d}, [(_src, _d)]))
    _q = 0
    for _grp in groups:
        for _src, _d in _grp[1]:
            _SLS[_t, _q] = _src
            _SLD[_t, _q] = _d
            _q += 1
        _q = ((_q + 15) // 16) * 16
    assert _q <= _MAXS

_CH2 = np.full((_T2P, 32), 32, np.int32)
for _v in _OUT_V:
    _K = _K2[_v]
    for _rl in range(_K * _K):
        _row = _TOFF2[_v] + _rl
        for _oo in range(32):
            _CH2[_row, _oo] = (_rl * 32 + _oo) // (_K * _K)

_XI_O = 0
_XM_O = _E1W
_AI_O = 2 * _E1W
_A2_O = 3 * _E1W
_SLS_O = _A2_O + _E2W
_SLD_O = _SLS_O + _MAXS
_CH_O = _SLD_O + _MAXS
_RTBL = _CH_O + _R2W * 32

_TBL = np.zeros((_NW, _RTBL), np.int32)
for _wid in range(_NW):
    _sid = _wid // 2
    _sl1 = slice(_sid * _E1W, (_sid + 1) * _E1W)
    _sl2 = slice(_wid * _E2W, (_wid + 1) * _E2W)
    _TBL[_wid, _XI_O:_XI_O + _E1W] = _XI[_sl1]
    _TBL[_wid, _XM_O:_XM_O + _E1W] = _XMF[_sl1].view(np.int32)
    _TBL[_wid, _AI_O:_AI_O + _E1W] = _AI[_sl1]
    _TBL[_wid, _A2_O:_A2_O + _E2W] = _A2I[_sl2]
    _TBL[_wid, _SLS_O:_SLS_O + _MAXS] = _SLS[_wid]
    _TBL[_wid, _SLD_O:_SLD_O + _MAXS] = _SLD[_wid]
    _cht = np.empty((_R2W // 16, 32, 16), np.int32)
    for _b in range(_R2W // 16):
        for _oo in range(32):
            for _i in range(16):
                _cht[_b, _oo, _i] = _CH2[_wid * _R2W + _b * 16 + _i, _oo]
    _TBL[_wid, _CH_O:_CH_O + _R2W * 32] = _cht.ravel()

_f32 = jnp.float32


def _sc_body(x_hbm, adj_hbm, scal_hbm, b1_hbm, w1_hbm, b2_hbm, w2_hbm,
             tbl_hbm, s_out,
             xv, adjv, scalv, b1r, w1r, b2r, w2r, tblv,
             h1c, h1ext, fl2, sacc, dsem, sh_h1):
    cid = lax.axis_index("c")
    sid = lax.axis_index("s")
    wid = sid * 2 + cid
    iot = lax.iota(jnp.int32, 16)

    copies = [
        pltpu.make_async_copy(x_hbm, xv, dsem),
        pltpu.make_async_copy(adj_hbm, adjv, dsem),
        pltpu.make_async_copy(scal_hbm, scalv, dsem),
        pltpu.make_async_copy(b1_hbm, b1r, dsem),
        pltpu.make_async_copy(w1_hbm, w1r, dsem),
        pltpu.make_async_copy(b2_hbm, b2r, dsem),
        pltpu.make_async_copy(w2_hbm, w2r, dsem),
        pltpu.make_async_copy(tbl_hbm.at[wid], tblv, dsem),
    ]
    for cp in copies:
        cp.start()
    for cp in copies:
        cp.wait()

    misc = scalv[...]
    lam1 = misc[0]
    lam2 = misc[1]
    b1v = b1r[...]
    w1v = [w1r[pl.ds(o * 16, 16)] for o in range(16)]

    def phase_a(b, carry):
        base = b * 256
        cols = []
        for c in range(16):
            ei = base + c + iot * 16
            xi = plsc.load_gather(tblv, [_XI_O + ei])
            xm = plsc.bitcast(plsc.load_gather(tblv, [_XM_O + ei]), _f32)
            ai = plsc.load_gather(tblv, [_AI_O + ei])
            xval = plsc.load_gather(xv, [xi])
            aval = plsc.load_gather(adjv, [ai])
            cols.append(xm * xval + lam1 * aval)
        for o in range(16):
            acc = cols[0] * w1v[o][0]
            for c in range(1, 16):
                acc = acc + cols[c] * w1v[o][c]
            val = jnp.maximum(acc + b1v[o], 0.0)
            plsc.store_scatter(h1c, [base + o + iot * 16], val)
        return carry

    lax.fori_loop(0, _R1W // 16, phase_a, None)

    pltpu.sync_copy(h1c, sh_h1.at[pl.ds(sid * _E1W, _E1W)])
    plsc.subcore_barrier()
    pltpu.sync_copy(sh_h1, h1ext.at[pl.ds(0, _E1)])
    h1ext[pl.ds(_E1, 16)] = jnp.zeros((16,), _f32)

    def phase_b_init(g, carry):
        e = g * 16 + iot
        a2 = plsc.load_gather(tblv, [_A2_O + e])
        plsc.store_scatter(fl2, [e], lam2 * plsc.load_gather(adjv, [a2]))
        return carry

    lax.fori_loop(0, _R2W, phase_b_init, None)
    fl2[pl.ds(_DUMP2, 16)] = jnp.zeros((16,), _f32)

    def phase_b_scat(q, carry):
        qq = q * 16 + iot
        src = plsc.load_gather(tblv, [_SLS_O + qq])
        dst = plsc.load_gather(tblv, [_SLD_O + qq])
        plsc.addupdate_scatter(fl2, [dst], plsc.load_gather(h1ext, [src]))
        return carry

    lax.fori_loop(0, _MAXS // 16, phase_b_scat, None)

    sacc[pl.ds(0, 16)] = jnp.zeros((16,), _f32)
    sacc[pl.ds(16, 16)] = jnp.zeros((16,), _f32)
    sacc[pl.ds(32, 16)] = jnp.zeros((16,), _f32)

    b2v = [b2r[pl.ds(0, 16)], b2r[pl.ds(16, 16)]]
    w2v = [w2r[pl.ds(o * 16, 16)] for o in range(32)]

    def phase_c(b, carry):
        base = b * 256
        cols = []
        for c in range(16):
            cols.append(plsc.load_gather(fl2, [base + c + iot * 16]))
        for o in range(32):
            acc = cols[0] * w2v[o][0]
            for c in range(1, 16):
                acc = acc + cols[c] * w2v[o][c]
            val = jnp.maximum(acc + b2v[o // 16][o % 16], 0.0)
            sidx = plsc.load_gather(tblv, [_CH_O + b * 512 + o * 16 + iot])
            plsc.addupdate_scatter(sacc, [sidx], val)
        return carry

    lax.fori_loop(0, _R2W // 16, phase_c, None)

    pltpu.sync_copy(sacc, s_out.at[wid])


def _tc_reduce(sp_ref, fcw_ref, fcb_ref, out_ref, g_ref):
    sp = sp_ref[...]
    stot = jnp.sum(sp, axis=0, keepdims=True)
    g_row = stot[:, 0:_C2]
    g_ref[...] = g_row
    prod = g_row * fcw_ref[...]
    out_ref[...] = jnp.sum(prod, axis=1, keepdims=True) + fcb_ref[...]


def kernel(x, adj, W1, b1, W2, b2, adj_lambda_1, adj_lambda_2, fc_w, fc_b):
    scal = jnp.concatenate([
        adj_lambda_1.reshape(-1), adj_lambda_2.reshape(-1),
        jnp.zeros((14,), _f32),
    ])
    mesh = plsc.VectorSubcoreMesh(core_axis_name="c", subcore_axis_name="s")
    sc = functools.partial(
        pl.kernel, _sc_body, mesh=mesh,
        compiler_params=pltpu.CompilerParams(needs_layout_passes=False),
        out_type=jax.ShapeDtypeStruct((_NW, 48), _f32),
        scratch_types=[
            pltpu.VMEM((1600,), _f32),
            pltpu.VMEM((_N * _N,), _f32),
            pltpu.VMEM((16,), _f32),
            pltpu.VMEM((16,), _f32),
            pltpu.VMEM((256,), _f32),
            pltpu.VMEM((32,), _f32),
            pltpu.VMEM((512,), _f32),
            pltpu.VMEM((_RTBL,), jnp.int32),
            pltpu.VMEM((_E1W,), _f32),
            pltpu.VMEM((_E1 + 16,), _f32),
            pltpu.VMEM((_E2W + 16,), _f32),
            pltpu.VMEM((48,), _f32),
            pltpu.SemaphoreType.DMA,
            pltpu.VMEM_SHARED((_E1,), _f32),
        ],
    )()
    s_part = sc(x.reshape(-1), adj.reshape(-1), scal,
                b1, W1.reshape(-1), b2, W2.reshape(-1), jnp.asarray(_TBL))

    out, g = pl.pallas_call(
        _tc_reduce,
        out_shape=[jax.ShapeDtypeStruct((1, 1), _f32),
                   jax.ShapeDtypeStruct((1, _C2), _f32)],
    )(s_part, fc_w, fc_b.reshape(1, 1))
    return out, g


# speedup vs baseline: 1.1946x; 1.0219x over previous
import functools
import numpy as np
import jax
import jax.numpy as jnp
from jax import lax
from jax.experimental import pallas as pl
from jax.experimental.pallas import tpu as pltpu
from jax.experimental.pallas import tpu_sc as plsc

_N = 100
_LVLS = 3
_D0 = 16
_C1 = 16
_C2 = 32
_EDGE_P = 0.06


def _structure():
    rng = np.random.RandomState(0)
    A = rng.rand(_N, _N) < _EDGE_P
    A = np.triu(A, 1)
    A = A | A.T
    nbhd1 = [sorted(set([v]) | set(np.nonzero(A[v])[0].tolist()))
             for v in range(_N)]
    rf = [[[v] for v in range(_N)]]
    for _ in range(1, _LVLS):
        prev = rf[-1]
        cur = []
        for v in range(_N):
            s = set()
            for w in nbhd1[v]:
                s.update(prev[w])
            cur.append(sorted(s))
        rf.append(cur)
    return nbhd1, rf


_NBHD1, _RF = _structure()
_OUT_V = list(range(_LVLS))
_W_NEED = sorted(set().union(*[set(_NBHD1[v]) for v in _OUT_V]))
_K1 = {w: len(_RF[1][w]) for w in _W_NEED}
_K2 = {v: len(_RF[2][v]) for v in _OUT_V}

_T1 = sum(k * k for k in _K1.values())
_T2 = sum(K * K for K in _K2.values())
_NW = 32
_R1W = 96
_R2W = 96
_T1P = 16 * _R1W
_T2P = _NW * _R2W
_E1 = _T1P * 16
_E1W = _R1W * 16
_E2W = _R2W * 16
_SENT = _E1
_MAXS = 800
_DUMP2 = _E2W

_TOFF1 = {}
_o = 0
for _w in _W_NEED:
    _TOFF1[_w] = _o
    _o += _K1[_w] * _K1[_w]
_TOFF2 = {}
_o = 0
for _v in _OUT_V:
    _TOFF2[_v] = _o
    _o += _K2[_v] * _K2[_v]

_XI = np.zeros((_E1,), np.int32)
_XMF = np.zeros((_E1,), np.float32)
_AIR = np.zeros((_E1,), np.int32)
_AIC = np.zeros((_E1,), np.int32)
for _w in _W_NEED:
    _k = _K1[_w]
    _S = _RF[1][_w]
    _base = _TOFF1[_w] * 16
    for _m in range(16 * _k * _k):
        _e = _base + _m
        _c, _rem = divmod(_m, _k * _k)
        _i, _j = divmod(_rem, _k)
        _AIR[_e] = _S[_i]
        _AIC[_e] = _S[_j]
        if _i == _j:
            _XI[_e] = _S[_i] * 16 + _c
            _XMF[_e] = 1.0

_A2R = np.zeros((_T2P * 16,), np.int32)
_A2C = np.zeros((_T2P * 16,), np.int32)
_PAIRS = [[] for _ in range(_NW)]
for _v in _OUT_V:
    _K = _K2[_v]
    _S2 = _RF[2][_v]
    _pos2 = {u: i for i, u in enumerate(_S2)}
    _b2 = _TOFF2[_v] * 16
    for _m in range(16 * _K * _K):
        _e = _b2 + _m
        _c, _rem = divmod(_m, _K * _K)
        _I, _J = divmod(_rem, _K)
        _A2R[_e] = _S2[_I]
        _A2C[_e] = _S2[_J]
    for _w in _NBHD1[_v]:
        _k = _K1[_w]
        _S1 = _RF[1][_w]
        for _c in range(16):
            for _il in range(_k):
                for _jl in range(_k):
                    _m = _c * _K * _K + _pos2[_S1[_il]] * _K + _pos2[_S1[_jl]]
                    _e = _b2 + _m
                    _src = _TOFF1[_w] * 16 + _c * _k * _k + _il * _k + _jl
                    _PAIRS[_e // _E2W].append((_src, _e % _E2W))
                    _CNTCHK = None

_SLS = np.full((_NW, _MAXS), _SENT, np.int32)
_SLD = np.full((_NW, _MAXS), _DUMP2, np.int32)
for _t in range(_NW):
    groups = []
    for _src, _d in _PAIRS[_t]:
        for _grp in groups:
            if _d not in _grp[0] and len(_grp[1]) < 16:
                _grp[0].add(_d)
                _grp[1].append((_src, _d))
                break
        else:
            groups.append(({_d}, [(_src, _d)]))
    _q = 0
    for _grp in groups:
        for _src, _d in _grp[1]:
            _SLS[_t, _q] = _src
            _SLD[_t, _q] = _d
            _q += 1
        _q = ((_q + 15) // 16) * 16
    assert _q <= _MAXS

_CH2 = np.full((_T2P, 32), 32, np.int32)
for _v in _OUT_V:
    _K = _K2[_v]
    for _rl in range(_K * _K):
        _row = _TOFF2[_v] + _rl
        for _oo in range(32):
            _CH2[_row, _oo] = (_rl * 32 + _oo) // (_K * _K)

_XI_O = 0
_XM_O = _E1W
_AIR_O = 2 * _E1W
_AIC_O = 3 * _E1W
_A2R_O = 4 * _E1W
_A2C_O = _A2R_O + _E2W
_SLS_O = _A2C_O + _E2W
_SLD_O = _SLS_O + _MAXS
_CH_O = _SLD_O + _MAXS
_RTBL = _CH_O + _R2W * 32

_TBL = np.zeros((_NW, _RTBL), np.int32)
for _wid in range(_NW):
    _sid = _wid // 2
    _sl1 = slice(_sid * _E1W, (_sid + 1) * _E1W)
    _sl2 = slice(_wid * _E2W, (_wid + 1) * _E2W)
    _TBL[_wid, _XI_O:_XI_O + _E1W] = _XI[_sl1]
    _TBL[_wid, _XM_O:_XM_O + _E1W] = _XMF[_sl1].view(np.int32)
    _TBL[_wid, _AIR_O:_AIR_O + _E1W] = _AIR[_sl1]
    _TBL[_wid, _AIC_O:_AIC_O + _E1W] = _AIC[_sl1]
    _TBL[_wid, _A2R_O:_A2R_O + _E2W] = _A2R[_sl2]
    _TBL[_wid, _A2C_O:_A2C_O + _E2W] = _A2C[_sl2]
    _TBL[_wid, _SLS_O:_SLS_O + _MAXS] = _SLS[_wid]
    _TBL[_wid, _SLD_O:_SLD_O + _MAXS] = _SLD[_wid]
    _cht = np.empty((_R2W // 16, 32, 16), np.int32)
    for _b in range(_R2W // 16):
        for _oo in range(32):
            for _i in range(16):
                _cht[_b, _oo, _i] = _CH2[_wid * _R2W + _b * 16 + _i, _oo]
    _TBL[_wid, _CH_O:_CH_O + _R2W * 32] = _cht.ravel()

_f32 = jnp.float32


def _sc_body(x_hbm, adj_hbm, lam1_hbm, lam2_hbm, b1_hbm, w1_hbm, b2_hbm,
             w2_hbm, tbl_hbm, s_out,
             xv, adjv, l1b, l2b, b1r, w1r, b2r, w2r, tblv,
             h1c, h1ext, fl2, sacc, dsem, sh_h1):
    cid = lax.axis_index("c")
    sid = lax.axis_index("s")
    wid = sid * 2 + cid
    iot = lax.iota(jnp.int32, 16)

    copies = [
        pltpu.make_async_copy(x_hbm, xv, dsem),
        pltpu.make_async_copy(adj_hbm, adjv, dsem),
        pltpu.make_async_copy(lam1_hbm, l1b.at[pl.ds(0, 1)], dsem),
        pltpu.make_async_copy(lam2_hbm, l2b.at[pl.ds(0, 1)], dsem),
        pltpu.make_async_copy(b1_hbm, b1r, dsem),
        pltpu.make_async_copy(w1_hbm, w1r, dsem),
        pltpu.make_async_copy(b2_hbm, b2r, dsem),
        pltpu.make_async_copy(w2_hbm, w2r, dsem),
        pltpu.make_async_copy(tbl_hbm.at[wid], tblv, dsem),
    ]
    for cp in copies:
        cp.start()
    for cp in copies:
        cp.wait()

    lam1 = l1b[...][0]
    lam2 = l2b[...][0]
    b1v = b1r[...]
    w1v = [plsc.load_gather(w1r, [iot * 0 + o, iot]) for o in range(16)]

    def phase_a(b, carry):
        base = b * 256
        cols = []
        for c in range(16):
            ei = base + c + iot * 16
            xi = plsc.load_gather(tblv, [_XI_O + ei])
            xm = plsc.bitcast(plsc.load_gather(tblv, [_XM_O + ei]), _f32)
            ar = plsc.load_gather(tblv, [_AIR_O + ei])
            ac = plsc.load_gather(tblv, [_AIC_O + ei])
            xval = plsc.load_gather(xv, [xi // 16, xi % 16])
            aval = plsc.load_gather(adjv, [ar, ac])
            cols.append(xm * xval + lam1 * aval)
        for o in range(16):
            acc = cols[0] * w1v[o][0]
            for c in range(1, 16):
                acc = acc + cols[c] * w1v[o][c]
            val = jnp.maximum(acc + b1v[o], 0.0)
            plsc.store_scatter(h1c, [base + o + iot * 16], val)
        return carry

    lax.fori_loop(0, _R1W // 16, phase_a, None)

    pltpu.sync_copy(h1c, sh_h1.at[pl.ds(sid * _E1W, _E1W)])
    plsc.subcore_barrier()
    pltpu.sync_copy(sh_h1, h1ext.at[pl.ds(0, _E1)])
    h1ext[pl.ds(_E1, 16)] = jnp.zeros((16,), _f32)

    def phase_b_init(g, carry):
        e = g * 16 + iot
        a2r = plsc.load_gather(tblv, [_A2R_O + e])
        a2c = plsc.load_gather(tblv, [_A2C_O + e])
        plsc.store_scatter(fl2, [e],
                           lam2 * plsc.load_gather(adjv, [a2r, a2c]))
        return carry

    lax.fori_loop(0, _R2W, phase_b_init, None)
    fl2[pl.ds(_DUMP2, 16)] = jnp.zeros((16,), _f32)

    def phase_b_scat(q, carry):
        qq = q * 16 + iot
        src = plsc.load_gather(tblv, [_SLS_O + qq])
        dst = plsc.load_gather(tblv, [_SLD_O + qq])
        plsc.addupdate_scatter(fl2, [dst], plsc.load_gather(h1ext, [src]))
        return carry

    lax.fori_loop(0, _MAXS // 16, phase_b_scat, None)

    sacc[pl.ds(0, 16)] = jnp.zeros((16,), _f32)
    sacc[pl.ds(16, 16)] = jnp.zeros((16,), _f32)
    sacc[pl.ds(32, 16)] = jnp.zeros((16,), _f32)

    b2v = [b2r[pl.ds(0, 16)], b2r[pl.ds(16, 16)]]
    w2v = [plsc.load_gather(w2r, [iot * 0 + o, iot]) for o in range(32)]

    def phase_c(b, carry):
        base = b * 256
        cols = []
        for c in range(16):
            cols.append(plsc.load_gather(fl2, [base + c + iot * 16]))
        for o in range(32):
            acc = cols[0] * w2v[o][0]
            for c in range(1, 16):
                acc = acc + cols[c] * w2v[o][c]
            val = jnp.maximum(acc + b2v[o // 16][o % 16], 0.0)
            sidx = plsc.load_gather(tblv, [_CH_O + b * 512 + o * 16 + iot])
            plsc.addupdate_scatter(sacc, [sidx], val)
        return carry

    lax.fori_loop(0, _R2W // 16, phase_c, None)

    pltpu.sync_copy(sacc, s_out.at[wid])


def _tc_reduce(sp_ref, fcw_ref, fcb_ref, out_ref, g_ref):
    sp = sp_ref[...]
    stot = jnp.sum(sp, axis=0, keepdims=True)
    g_row = stot[:, 0:_C2]
    g_ref[...] = g_row
    prod = g_row * fcw_ref[...]
    out_ref[...] = jnp.sum(prod, axis=1, keepdims=True) + fcb_ref[...]


def kernel(x, adj, W1, b1, W2, b2, adj_lambda_1, adj_lambda_2, fc_w, fc_b):
    mesh = plsc.VectorSubcoreMesh(core_axis_name="c", subcore_axis_name="s")
    sc = functools.partial(
        pl.kernel, _sc_body, mesh=mesh,
        compiler_params=pltpu.CompilerParams(needs_layout_passes=False),
        out_type=jax.ShapeDtypeStruct((_NW, 48), _f32),
        scratch_types=[
            pltpu.VMEM((_N, 16), _f32),
            pltpu.VMEM((_N, _N), _f32),
            pltpu.VMEM((16,), _f32),
            pltpu.VMEM((16,), _f32),
            pltpu.VMEM((16,), _f32),
            pltpu.VMEM((16, 16), _f32),
            pltpu.VMEM((32,), _f32),
            pltpu.VMEM((32, 16), _f32),
            pltpu.VMEM((_RTBL,), jnp.int32),
            pltpu.VMEM((_E1W,), _f32),
            pltpu.VMEM((_E1 + 16,), _f32),
            pltpu.VMEM((_E2W + 16,), _f32),
            pltpu.VMEM((48,), _f32),
            pltpu.SemaphoreType.DMA,
            pltpu.VMEM_SHARED((_E1,), _f32),
        ],
    )()
    s_part = sc(x, adj, adj_lambda_1, adj_lambda_2,
                b1, W1, b2, W2, jnp.asarray(_TBL))

    out, g = pl.pallas_call(
        _tc_reduce,
        out_shape=[jax.ShapeDtypeStruct((1, 1), _f32),
                   jax.ShapeDtypeStruct((1, _C2), _f32)],
    )(s_part, fc_w, fc_b.reshape(1, 1))
    return out, g


# speedup vs baseline: 1.1960x; 1.0012x over previous
import functools
import numpy as np
import jax
import jax.numpy as jnp
from jax import lax
from jax.experimental import pallas as pl
from jax.experimental.pallas import tpu as pltpu
from jax.experimental.pallas import tpu_sc as plsc

_N = 100
_LVLS = 3
_D0 = 16
_C1 = 16
_C2 = 32
_EDGE_P = 0.06


def _structure():
    rng = np.random.RandomState(0)
    A = rng.rand(_N, _N) < _EDGE_P
    A = np.triu(A, 1)
    A = A | A.T
    nbhd1 = [sorted(set([v]) | set(np.nonzero(A[v])[0].tolist()))
             for v in range(_N)]
    rf = [[[v] for v in range(_N)]]
    for _ in range(1, _LVLS):
        prev = rf[-1]
        cur = []
        for v in range(_N):
            s = set()
            for w in nbhd1[v]:
                s.update(prev[w])
            cur.append(sorted(s))
        rf.append(cur)
    return nbhd1, rf


_NBHD1, _RF = _structure()
_OUT_V = list(range(_LVLS))
_W_NEED = sorted(set().union(*[set(_NBHD1[v]) for v in _OUT_V]))
_K1 = {w: len(_RF[1][w]) for w in _W_NEED}
_K2 = {v: len(_RF[2][v]) for v in _OUT_V}

_T1 = sum(k * k for k in _K1.values())
_T2 = sum(K * K for K in _K2.values())
_NW = 32
_R1W = 96
_R2W = 96
_T1P = 16 * _R1W
_T2P = _NW * _R2W
_E1 = _T1P * 16
_E1W = _R1W * 16
_E2W = _R2W * 16
_SENT = _E1
_MAXS = 800
_DUMP2 = _E2W

_TOFF1 = {}
_o = 0
for _w in _W_NEED:
    _TOFF1[_w] = _o
    _o += _K1[_w] * _K1[_w]
_TOFF2 = {}
_o = 0
for _v in _OUT_V:
    _TOFF2[_v] = _o
    _o += _K2[_v] * _K2[_v]

_XI = np.zeros((_E1,), np.int32)
_XMF = np.zeros((_E1,), np.float32)
_AIR = np.zeros((_E1,), np.int32)
_AIC = np.zeros((_E1,), np.int32)
for _w in _W_NEED:
    _k = _K1[_w]
    _S = _RF[1][_w]
    _base = _TOFF1[_w] * 16
    for _m in range(16 * _k * _k):
        _e = _base + _m
        _c, _rem = divmod(_m, _k * _k)
        _i, _j = divmod(_rem, _k)
        _AIR[_e] = _S[_i]
        _AIC[_e] = _S[_j]
        if _i == _j:
            _XI[_e] = _S[_i] * 16 + _c
            _XMF[_e] = 1.0

_A2R = np.zeros((_T2P * 16,), np.int32)
_A2C = np.zeros((_T2P * 16,), np.int32)
_PAIRS = [[] for _ in range(_NW)]
for _v in _OUT_V:
    _K = _K2[_v]
    _S2 = _RF[2][_v]
    _pos2 = {u: i for i, u in enumerate(_S2)}
    _b2 = _TOFF2[_v] * 16
    for _m in range(16 * _K * _K):
        _e = _b2 + _m
        _c, _rem = divmod(_m, _K * _K)
        _I, _J = divmod(_rem, _K)
        _A2R[_e] = _S2[_I]
        _A2C[_e] = _S2[_J]
    for _w in _NBHD1[_v]:
        _k = _K1[_w]
        _S1 = _RF[1][_w]
        for _c in range(16):
            for _il in range(_k):
                for _jl in range(_k):
                    _m = _c * _K * _K + _pos2[_S1[_il]] * _K + _pos2[_S1[_jl]]
                    _e = _b2 + _m
                    _src = _TOFF1[_w] * 16 + _c * _k * _k + _il * _k + _jl
                    _PAIRS[_e // _E2W].append((_src, _e % _E2W))

_SLS = np.full((_NW, _MAXS), _SENT, np.int32)
_SLD = np.full((_NW, _MAXS), _DUMP2, np.int32)
for _t in range(_NW):
    groups = []
    for _src, _d in _PAIRS[_t]:
        for _grp in groups:
            if _d not in _grp[0] and len(_grp[1]) < 16:
                _grp[0].add(_d)
                _grp[1].append((_src, _d))
                break
        else:
            groups.append(({_d}, [(_src, _d)]))
    _q = 0
    for _grp in groups:
        for _src, _d in _grp[1]:
            _SLS[_t, _q] = _src
            _SLD[_t, _q] = _d
            _q += 1
        _q = ((_q + 15) // 16) * 16
    assert _q <= _MAXS

_CH2 = np.full((_T2P, 32), 32, np.int32)
for _v in _OUT_V:
    _K = _K2[_v]
    for _rl in range(_K * _K):
        _row = _TOFF2[_v] + _rl
        for _oo in range(32):
            _CH2[_row, _oo] = (_rl * 32 + _oo) // (_K * _K)

_XI_O = 0
_XM_O = _E1W
_AIR_O = 2 * _E1W
_AIC_O = 3 * _E1W
_A2R_O = 4 * _E1W
_A2C_O = _A2R_O + _E2W
_SLS_O = _A2C_O + _E2W
_SLD_O = _SLS_O + _MAXS
_CH_O = _SLD_O + _MAXS
_RTBL = _CH_O + _R2W * 32

_TBL = np.zeros((_NW, _RTBL), np.int32)
for _wid in range(_NW):
    _sid = _wid // 2
    _sl1 = slice(_sid * _E1W, (_sid + 1) * _E1W)
    _sl2 = slice(_wid * _E2W, (_wid + 1) * _E2W)
    _TBL[_wid, _XI_O:_XI_O + _E1W] = _XI[_sl1]
    _TBL[_wid, _XM_O:_XM_O + _E1W] = _XMF[_sl1].view(np.int32)
    _TBL[_wid, _AIR_O:_AIR_O + _E1W] = _AIR[_sl1]
    _TBL[_wid, _AIC_O:_AIC_O + _E1W] = _AIC[_sl1]
    _TBL[_wid, _A2R_O:_A2R_O + _E2W] = _A2R[_sl2]
    _TBL[_wid, _A2C_O:_A2C_O + _E2W] = _A2C[_sl2]
    _TBL[_wid, _SLS_O:_SLS_O + _MAXS] = _SLS[_wid]
    _TBL[_wid, _SLD_O:_SLD_O + _MAXS] = _SLD[_wid]
    _cht = np.empty((_R2W // 16, 32, 16), np.int32)
    for _b in range(_R2W // 16):
        for _oo in range(32):
            for _i in range(16):
                _cht[_b, _oo, _i] = _CH2[_wid * _R2W + _b * 16 + _i, _oo]
    _TBL[_wid, _CH_O:_CH_O + _R2W * 32] = _cht.ravel()

_f32 = jnp.float32


def _sc_body(x_hbm, adj_hbm, lam1_hbm, lam2_hbm, b1_hbm, w1_hbm, b2_hbm,
             w2_hbm, tbl_hbm, s_out,
             xv, adjv, l1b, l2b, b1r, w1r, b2r, w2r, tblv,
             h1c, h1ext, fl2, sacc, dsem, sh_h1):
    cid = lax.axis_index("c")
    sid = lax.axis_index("s")
    wid = sid * 2 + cid
    iot = lax.iota(jnp.int32, 16)

    copies = [
        pltpu.make_async_copy(x_hbm, xv, dsem),
        pltpu.make_async_copy(adj_hbm, adjv, dsem),
        pltpu.make_async_copy(lam1_hbm, l1b.at[pl.ds(0, 1)], dsem),
        pltpu.make_async_copy(lam2_hbm, l2b.at[pl.ds(0, 1)], dsem),
        pltpu.make_async_copy(b1_hbm, b1r, dsem),
        pltpu.make_async_copy(w1_hbm, w1r, dsem),
        pltpu.make_async_copy(b2_hbm, b2r, dsem),
        pltpu.make_async_copy(w2_hbm, w2r, dsem),
        pltpu.make_async_copy(tbl_hbm.at[wid], tblv, dsem),
    ]
    for cp in copies:
        cp.start()
    for cp in copies:
        cp.wait()

    lam1 = l1b[...][0]
    lam2 = l2b[...][0]
    b1v = b1r[...]
    w1v = [plsc.load_gather(w1r, [iot * 0 + o, iot]) for o in range(16)]

    def phase_a(b, carry):
        base = b * 256
        cols = []
        for c in range(16):
            ei = base + c + iot * 16
            xi = plsc.load_gather(tblv, [_XI_O + ei])
            xm = plsc.bitcast(plsc.load_gather(tblv, [_XM_O + ei]), _f32)
            ar = plsc.load_gather(tblv, [_AIR_O + ei])
            ac = plsc.load_gather(tblv, [_AIC_O + ei])
            xval = plsc.load_gather(xv, [xi // 16, xi % 16])
            aval = plsc.load_gather(adjv, [ar, ac])
            cols.append(xm * xval + lam1 * aval)
        for o in range(16):
            acc0 = cols[0] * w1v[o][0]
            acc1 = cols[1] * w1v[o][1]
            for c in range(2, 16, 2):
                acc0 = acc0 + cols[c] * w1v[o][c]
                acc1 = acc1 + cols[c + 1] * w1v[o][c + 1]
            val = jnp.maximum(acc0 + acc1 + b1v[o], 0.0)
            plsc.store_scatter(h1c, [base + o + iot * 16], val)
        return carry

    lax.fori_loop(0, _R1W // 16, phase_a, None)

    pltpu.sync_copy(h1c, sh_h1.at[pl.ds(sid * _E1W, _E1W)])
    plsc.subcore_barrier()
    cp_down = pltpu.make_async_copy(sh_h1, h1ext.at[pl.ds(0, _E1)], dsem)
    cp_down.start()

    def phase_b_init(g, carry):
        e = g * 16 + iot
        a2r = plsc.load_gather(tblv, [_A2R_O + e])
        a2c = plsc.load_gather(tblv, [_A2C_O + e])
        plsc.store_scatter(fl2, [e],
                           lam2 * plsc.load_gather(adjv, [a2r, a2c]))
        return carry

    lax.fori_loop(0, _R2W, phase_b_init, None)
    fl2[pl.ds(_DUMP2, 16)] = jnp.zeros((16,), _f32)
    cp_down.wait()
    h1ext[pl.ds(_E1, 16)] = jnp.zeros((16,), _f32)

    def phase_b_scat(q, carry):
        qq = q * 16 + iot
        src = plsc.load_gather(tblv, [_SLS_O + qq])
        dst = plsc.load_gather(tblv, [_SLD_O + qq])
        plsc.addupdate_scatter(fl2, [dst], plsc.load_gather(h1ext, [src]))
        return carry

    lax.fori_loop(0, _MAXS // 16, phase_b_scat, None)

    sacc[pl.ds(0, 16)] = jnp.zeros((16,), _f32)
    sacc[pl.ds(16, 16)] = jnp.zeros((16,), _f32)
    sacc[pl.ds(32, 16)] = jnp.zeros((16,), _f32)

    b2v = [b2r[pl.ds(0, 16)], b2r[pl.ds(16, 16)]]
    w2v = [plsc.load_gather(w2r, [iot * 0 + o, iot]) for o in range(32)]

    def phase_c(b, carry):
        base = b * 256
        cols = []
        for c in range(16):
            cols.append(plsc.load_gather(fl2, [base + c + iot * 16]))
        for o in range(32):
            acc0 = cols[0] * w2v[o][0]
            acc1 = cols[1] * w2v[o][1]
            for c in range(2, 16, 2):
                acc0 = acc0 + cols[c] * w2v[o][c]
                acc1 = acc1 + cols[c + 1] * w2v[o][c + 1]
            val = jnp.maximum(acc0 + acc1 + b2v[o // 16][o % 16], 0.0)
            sidx = plsc.load_gather(tblv, [_CH_O + b * 512 + o * 16 + iot])
            plsc.addupdate_scatter(sacc, [sidx], val)
        return carry

    lax.fori_loop(0, _R2W // 16, phase_c, None)

    pltpu.sync_copy(sacc, s_out.at[wid])


def _tc_reduce(sp_ref, fcw_ref, fcb_ref, out_ref, g_ref):
    sp = sp_ref[...]
    stot = jnp.sum(sp, axis=0, keepdims=True)
    g_row = stot[:, 0:_C2]
    g_ref[...] = g_row
    prod = g_row * fcw_ref[...]
    out_ref[...] = jnp.sum(prod, axis=1, keepdims=True) + fcb_ref[...]


def kernel(x, adj, W1, b1, W2, b2, adj_lambda_1, adj_lambda_2, fc_w, fc_b):
    mesh = plsc.VectorSubcoreMesh(core_axis_name="c", subcore_axis_name="s")
    sc = functools.partial(
        pl.kernel, _sc_body, mesh=mesh,
        compiler_params=pltpu.CompilerParams(needs_layout_passes=False),
        out_type=jax.ShapeDtypeStruct((_NW, 48), _f32),
        scratch_types=[
            pltpu.VMEM((_N, 16), _f32),
            pltpu.VMEM((_N, _N), _f32),
            pltpu.VMEM((16,), _f32),
            pltpu.VMEM((16,), _f32),
            pltpu.VMEM((16,), _f32),
            pltpu.VMEM((16, 16), _f32),
            pltpu.VMEM((32,), _f32),
            pltpu.VMEM((32, 16), _f32),
            pltpu.VMEM((_RTBL,), jnp.int32),
            pltpu.VMEM((_E1W,), _f32),
            pltpu.VMEM((_E1 + 16,), _f32),
            pltpu.VMEM((_E2W + 16,), _f32),
            pltpu.VMEM((48,), _f32),
            pltpu.SemaphoreType.DMA,
            pltpu.VMEM_SHARED((_E1,), _f32),
        ],
    )()
    s_part = sc(x, adj, adj_lambda_1, adj_lambda_2,
                b1, W1, b2, W2, jnp.asarray(_TBL))

    out, g = pl.pallas_call(
        _tc_reduce,
        out_shape=[jax.ShapeDtypeStruct((1, 1), _f32),
                   jax.ShapeDtypeStruct((1, _C2), _f32)],
    )(s_part, fc_w, fc_b.reshape(1, 1))
    return out, g
